# baseline (speedup 1.0000x reference)
import functools

import jax
import jax.numpy as jnp
from jax import lax
from jax.experimental import pallas as pl
from jax.experimental.pallas import tpu as pltpu
from jax.experimental.pallas import tpu_sc as plsc

_NC = 2
_NS = 16
_NW = _NC * _NS
_RB = 8
_PS = 56
_PD = 128


_NSPLIT = 2


def kernel(token_ids, W):
    B, S = token_ids.shape
    dim = W.shape[1]
    step = B // _NSPLIT
    parts = []
    for h in range(_NSPLIT):
        padded = _gather_padded(
            lax.slice_in_dim(token_ids, h * step, (h + 1) * step), W)
        parts.append(padded[:, :S, :dim])
    return jnp.concatenate(parts, axis=0) if _NSPLIT > 1 else parts[0]


def _gather_padded(token_ids, W):
    B, S = token_ids.shape
    dim = W.shape[1]
    rows_per_w = B // _NW
    rounds = rows_per_w // _RB

    mesh = plsc.VectorSubcoreMesh(core_axis_name="c", subcore_axis_name="s")

    @functools.partial(
        pl.kernel,
        mesh=mesh,
        out_type=jax.ShapeDtypeStruct((B, _PS, _PD), W.dtype),
        scratch_types=[
            pltpu.VMEM((rows_per_w * S // 100, 100), jnp.int32),
            pltpu.VMEM((_RB * S, dim), jnp.float32),
            pltpu.VMEM((_RB * S, dim), jnp.float32),
            pltpu.SemaphoreType.DMA,
            pltpu.SemaphoreType.DMA,
            pltpu.SemaphoreType.DMA,
            pltpu.SemaphoreType.DMA,
        ],
        compiler_params=pltpu.CompilerParams(use_tc_tiling_on_sc=False),
    )
    def gather_kernel(w_hbm, i_hbm, o_hbm, idx_v, buf0, buf1,
                      gsem0, gsem1, wsem0, wsem1):
        wid = lax.axis_index("s") * _NC + lax.axis_index("c")
        base = wid * rows_per_w
        idx_rows = rows_per_w * S // 100
        pltpu.sync_copy(i_hbm.at[pl.ds(wid * idx_rows, idx_rows)], idx_v)

        bufs = (buf0, buf1)
        gsems = (gsem0, gsem1)
        wsems = (wsem0, wsem1)

        streams_per_round = _RB * S // 100

        def fire(r, slot):
            for k in range(streams_per_round):
                pltpu.async_copy(
                    w_hbm.at[idx_v.at[r * streams_per_round + k]],
                    bufs[slot].at[pl.ds(k * 100, 100)],
                    gsems[slot],
                )

        def drain_gathers(slot):
            pltpu.make_async_copy(
                w_hbm.at[pl.ds(0, _RB * S)],
                bufs[slot],
                gsems[slot],
            ).wait()

        def start_wb(r, slot):
            for b in range(_RB):
                pltpu.async_copy(
                    bufs[slot].at[pl.ds(b * S, S)],
                    o_hbm.at[base + r * _RB + b, pl.ds(0, S), pl.ds(0, dim)],
                    wsems[slot],
                )

        def drain_wb(slot):
            pltpu.make_async_copy(
                w_hbm.at[pl.ds(0, _RB * S)],
                bufs[slot],
                wsems[slot],
            ).wait()

        fire(0, 0)
        drain_gathers(0)
        fire(1, 1)
        start_wb(0, 0)
        drain_gathers(1)
        drain_wb(0)
        fire(2, 0)
        start_wb(1, 1)

        @pl.loop(2, rounds - 2, step=2)
        def _(j):
            drain_gathers(0)
            drain_wb(1)
            fire(j + 1, 1)
            start_wb(j, 0)
            drain_gathers(1)
            drain_wb(0)
            fire(j + 2, 0)
            start_wb(j + 1, 1)

        drain_gathers(0)
        drain_wb(1)
        fire(rounds - 1, 1)
        start_wb(rounds - 2, 0)
        drain_gathers(1)
        drain_wb(0)
        start_wb(rounds - 1, 1)
        drain_wb(1)

    return gather_kernel(W, token_ids.reshape(B * S // 100, 100))


# speedup vs baseline: 1.2924x; 1.2924x over previous
import functools

import jax
import jax.numpy as jnp
from jax import lax
from jax.experimental import pallas as pl
from jax.experimental.pallas import tpu as pltpu
from jax.experimental.pallas import tpu_sc as plsc

_NC = 2
_NS = 16
_NW = _NC * _NS
_RB = 8
_PS = 56
_PD = 128


def kernel(token_ids, W):
    B, S = token_ids.shape
    dim = W.shape[1]
    padded = _gather_padded(token_ids, W)
    return padded[:, :S, :dim]


def _gather_padded(token_ids, W):
    B, S = token_ids.shape
    dim = W.shape[1]
    rows_per_w = B // _NW
    rounds = rows_per_w // _RB

    mesh = plsc.VectorSubcoreMesh(core_axis_name="c", subcore_axis_name="s")

    @functools.partial(
        pl.kernel,
        mesh=mesh,
        out_type=jax.ShapeDtypeStruct((B, _PS, _PD), W.dtype),
        scratch_types=[
            pltpu.VMEM((rows_per_w * S // 100, 100), jnp.int32),
            pltpu.VMEM((_RB * S, dim), jnp.float32),
            pltpu.VMEM((_RB * S, dim), jnp.float32),
            pltpu.VMEM((_RB * S, dim), jnp.float32),
            pltpu.VMEM((_RB * S, dim), jnp.float32),
            pltpu.SemaphoreType.DMA,
            pltpu.SemaphoreType.DMA,
            pltpu.SemaphoreType.DMA,
            pltpu.SemaphoreType.DMA,
            pltpu.SemaphoreType.DMA,
            pltpu.SemaphoreType.DMA,
            pltpu.SemaphoreType.DMA,
            pltpu.SemaphoreType.DMA,
        ],
        compiler_params=pltpu.CompilerParams(use_tc_tiling_on_sc=False),
    )
    def gather_kernel(w_hbm, i_hbm, o_hbm, idx_v,
                      bufa0, bufa1, bufb0, bufb1,
                      gsa0, gsa1, gsb0, gsb1, wsa0, wsa1, wsb0, wsb1):
        wid = lax.axis_index("s") * _NC + lax.axis_index("c")
        base = wid * rows_per_w
        idx_rows = rows_per_w * S // 100
        pltpu.sync_copy(i_hbm.at[pl.ds(wid * idx_rows, idx_rows)], idx_v)

        half_rounds = rounds // 2
        spr = _RB * S // 100
        pipes = (
            dict(bufs=(bufa0, bufa1), gs=(gsa0, gsa1), ws=(wsa0, wsa1),
                 row0=base, idx0=wid * idx_rows * 0 + 0),
            dict(bufs=(bufb0, bufb1), gs=(gsb0, gsb1), ws=(wsb0, wsb1),
                 row0=base + half_rounds * _RB, idx0=half_rounds * spr),
        )

        def fire(p, r, slot):
            pp = pipes[p]
            for k in range(spr):
                pltpu.async_copy(
                    w_hbm.at[idx_v.at[pp["idx0"] + r * spr + k]],
                    pp["bufs"][slot].at[pl.ds(k * 100, 100)],
                    pp["gs"][slot],
                )

        def drain_g(p, slot):
            pp = pipes[p]
            pltpu.make_async_copy(
                w_hbm.at[pl.ds(0, _RB * S)], pp["bufs"][slot], pp["gs"][slot]
            ).wait()

        def start_wb(p, r, slot):
            pp = pipes[p]
            for b in range(_RB):
                pltpu.async_copy(
                    pp["bufs"][slot].at[pl.ds(b * S, S)],
                    o_hbm.at[pp["row0"] + r * _RB + b,
                             pl.ds(0, S), pl.ds(0, dim)],
                    pp["ws"][slot],
                )

        def drain_wb(p, slot):
            pp = pipes[p]
            pltpu.make_async_copy(
                w_hbm.at[pl.ds(0, _RB * S)], pp["bufs"][slot], pp["ws"][slot]
            ).wait()

        for p in (0, 1):
            fire(p, 0, 0)
        for p in (0, 1):
            drain_g(p, 0)
            fire(p, 1, 1)
            start_wb(p, 0, 0)
        for p in (0, 1):
            drain_g(p, 1)
            drain_wb(p, 0)
            fire(p, 2, 0)
            start_wb(p, 1, 1)

        @pl.loop(2, half_rounds - 2, step=2)
        def _(j):
            for p in (0, 1):
                drain_g(p, 0)
                drain_wb(p, 1)
                fire(p, j + 1, 1)
                start_wb(p, j, 0)
            for p in (0, 1):
                drain_g(p, 1)
                drain_wb(p, 0)
                fire(p, j + 2, 0)
                start_wb(p, j + 1, 1)

        for p in (0, 1):
            drain_g(p, 0)
            drain_wb(p, 1)
            fire(p, half_rounds - 1, 1)
            start_wb(p, half_rounds - 2, 0)
        for p in (0, 1):
            drain_g(p, 1)
            drain_wb(p, 0)
            start_wb(p, half_rounds - 1, 1)
            drain_wb(p, 1)

    return gather_kernel(W, token_ids.reshape(B * S // 100, 100))


# speedup vs baseline: 1.2945x; 1.0016x over previous
import functools

import jax
import jax.numpy as jnp
from jax import lax
from jax.experimental import pallas as pl
from jax.experimental.pallas import tpu as pltpu
from jax.experimental.pallas import tpu_sc as plsc

_NC = 2
_NS = 16
_NW = _NC * _NS
_RB = 8


def kernel(token_ids, W):
    B, S = token_ids.shape
    dim = W.shape[1]
    padded = _gather_padded(token_ids, W)
    return padded[:, :S, :dim]


def _gather_padded(token_ids, W):
    B, S = token_ids.shape
    dim = W.shape[1]
    ps = -(-S // 8) * 8
    pd = 128
    rows_per_w = B // _NW
    rounds = rows_per_w // _RB

    mesh = plsc.VectorSubcoreMesh(core_axis_name="c", subcore_axis_name="s")

    @functools.partial(
        pl.kernel,
        mesh=mesh,
        out_type=jax.ShapeDtypeStruct((B, ps, pd), W.dtype),
        scratch_types=[
            pltpu.VMEM((rows_per_w * S // 100, 100), jnp.int32),
            pltpu.VMEM((_RB * S, dim), jnp.float32),
            pltpu.VMEM((_RB * S, dim), jnp.float32),
            pltpu.VMEM((_RB * S, dim), jnp.float32),
            pltpu.VMEM((_RB * S, dim), jnp.float32),
            pltpu.SemaphoreType.DMA,
            pltpu.SemaphoreType.DMA,
            pltpu.SemaphoreType.DMA,
            pltpu.SemaphoreType.DMA,
            pltpu.SemaphoreType.DMA,
            pltpu.SemaphoreType.DMA,
            pltpu.SemaphoreType.DMA,
            pltpu.SemaphoreType.DMA,
        ],
        compiler_params=pltpu.CompilerParams(use_tc_tiling_on_sc=False),
    )
    def gather_kernel(w_hbm, i_hbm, o_hbm, idx_v,
                      bufa0, bufa1, bufb0, bufb1,
                      gsa0, gsa1, gsb0, gsb1, wsa0, wsa1, wsb0, wsb1):
        wid = lax.axis_index("s") * _NC + lax.axis_index("c")
        base = wid * rows_per_w
        idx_rows = rows_per_w * S // 100
        pltpu.sync_copy(i_hbm.at[pl.ds(wid * idx_rows, idx_rows)], idx_v)

        half_rounds = rounds // 2
        spr = _RB * S // 100
        pipes = (
            dict(bufs=(bufa0, bufa1), gs=(gsa0, gsa1), ws=(wsa0, wsa1),
                 row0=base, idx0=0),
            dict(bufs=(bufb0, bufb1), gs=(gsb0, gsb1), ws=(wsb0, wsb1),
                 row0=base + half_rounds * _RB, idx0=half_rounds * spr),
        )

        def fire(p, r, slot):
            pp = pipes[p]
            for k in range(spr):
                pltpu.async_copy(
                    w_hbm.at[idx_v.at[pp["idx0"] + r * spr + k]],
                    pp["bufs"][slot].at[pl.ds(k * 100, 100)],
                    pp["gs"][slot],
                )

        def drain_g(p, slot):
            pp = pipes[p]
            pltpu.make_async_copy(
                w_hbm.at[pl.ds(0, _RB * S)], pp["bufs"][slot], pp["gs"][slot]
            ).wait()

        def start_wb(p, r, slot):
            pp = pipes[p]
            for b in range(_RB):
                pltpu.async_copy(
                    pp["bufs"][slot].at[pl.ds(b * S, S)],
                    o_hbm.at[pp["row0"] + r * _RB + b,
                             pl.ds(0, S), pl.ds(0, dim)],
                    pp["ws"][slot],
                )

        def drain_wb(p, slot):
            pp = pipes[p]
            pltpu.make_async_copy(
                w_hbm.at[pl.ds(0, _RB * S)], pp["bufs"][slot], pp["ws"][slot]
            ).wait()

        for p in (0, 1):
            fire(p, 0, 0)
        for p in (0, 1):
            drain_g(p, 0)
            fire(p, 1, 1)
            start_wb(p, 0, 0)
        for p in (0, 1):
            drain_g(p, 1)
            drain_wb(p, 0)
            fire(p, 2, 0)
            start_wb(p, 1, 1)

        @pl.loop(2, half_rounds - 2, step=2)
        def _(j):
            for p in (0, 1):
                drain_g(p, 0)
                drain_wb(p, 1)
                fire(p, j + 1, 1)
                start_wb(p, j, 0)
            for p in (0, 1):
                drain_g(p, 1)
                drain_wb(p, 0)
                fire(p, j + 2, 0)
                start_wb(p, j + 1, 1)

        for p in (0, 1):
            drain_g(p, 0)
            drain_wb(p, 1)
            fire(p, half_rounds - 1, 1)
            start_wb(p, half_rounds - 2, 0)
        for p in (0, 1):
            drain_g(p, 1)
            drain_wb(p, 0)
            start_wb(p, half_rounds - 1, 1)
            drain_wb(p, 1)

    return gather_kernel(W, token_ids.reshape(B * S // 100, 100))
